# Initial kernel scaffold; baseline (speedup 1.0000x reference)
import jax
import jax.numpy as jnp
from jax.experimental import pallas as pl


def kernel(x, edge_index, W1a, b1a, W1b, b1b, W2a, b2a, W2b, b2b, Wfc, bfc):
    raise NotImplementedError("write your pallas kernel here")


# speedup vs baseline: 4.5801x; 4.5801x over previous
import functools

import jax
import jax.numpy as jnp
from jax import lax
from jax.experimental import pallas as pl
from jax.experimental.pallas import tpu as pltpu
from jax.experimental.pallas import tpu_sc as plsc

N_NODES = 10000
N_EDGES = 320000
D_FEAT = 128
HIDDEN = 128
N_CLASSES = 64

NC = 2
NS = 16
NW = NC * NS
EPW = N_EDGES // NW
CE = 80
NCHUNK = EPW // CE
N_PAD = 10240
STRIPE = N_PAD // NS
ZB = 128

_sc_mesh = plsc.VectorSubcoreMesh(
    core_axis_name="c", subcore_axis_name="s", num_cores=NC, num_subcores=NS
)


@functools.partial(
    pl.kernel,
    out_type=jax.ShapeDtypeStruct((NC, N_PAD, D_FEAT), jnp.float32),
    mesh=_sc_mesh,
    scratch_types=[
        pltpu.VMEM((CE,), jnp.int32),
        pltpu.VMEM((CE,), jnp.int32),
        pltpu.VMEM((CE, D_FEAT), jnp.float32),
        pltpu.VMEM((ZB, D_FEAT), jnp.float32),
        pltpu.VMEM_SHARED((N_PAD, D_FEAT), jnp.float32),
        pltpu.SemaphoreType.DMA,
    ],
)
def _sc_aggregate(x_hbm, src_hbm, dst_hbm, out_hbm, src_v, dst_v, rows_v,
                  zero_v, aggr_sh, sem):
    c = lax.axis_index("c")
    s = lax.axis_index("s")
    wid = c * NS + s

    zvec = jnp.zeros((16,), jnp.float32)

    def _zb_body(r, carry):
        for j in range(D_FEAT // 16):
            zero_v[r, pl.ds(j * 16, 16)] = zvec
        return carry

    lax.fori_loop(0, ZB, _zb_body, 0)
    row0 = s * STRIPE
    for k in range(STRIPE // ZB):
        pltpu.sync_copy(zero_v, aggr_sh.at[pl.ds(row0 + k * ZB, ZB)])
    plsc.subcore_barrier()

    ebase = wid * EPW

    def _edge_body(i, carry):
        off = ebase + i * CE
        pltpu.sync_copy(src_hbm.at[pl.ds(off, CE)], src_v)
        pltpu.sync_copy(dst_hbm.at[pl.ds(off, CE)], dst_v)
        pltpu.async_copy(x_hbm.at[src_v], rows_v, sem).wait()
        pltpu.sync_copy(rows_v, aggr_sh.at[dst_v], add=True)
        return carry

    lax.fori_loop(0, NCHUNK, _edge_body, 0)
    plsc.subcore_barrier()

    pltpu.sync_copy(
        aggr_sh.at[pl.ds(row0, STRIPE)], out_hbm.at[c, pl.ds(row0, STRIPE)]
    )


BLK = 1000


def _mlp1_body(x_ref, a0_ref, a1_ref, wa_ref, ba_ref, wb_ref, bb_ref, o_ref):
    t = x_ref[...] + a0_ref[...] + a1_ref[...]
    h = jnp.dot(t, wa_ref[...], preferred_element_type=jnp.float32) + ba_ref[...]
    h = jnp.maximum(h, 0.0)
    h = jnp.dot(h, wb_ref[...], preferred_element_type=jnp.float32) + bb_ref[...]
    o_ref[...] = jnp.maximum(h, 0.0)


def _mlp2_body(h_ref, a0_ref, a1_ref, wa_ref, ba_ref, wb_ref, bb_ref,
               wfc_ref, bfc_ref, o_ref):
    t = h_ref[...] + a0_ref[...] + a1_ref[...]
    u = jnp.dot(t, wa_ref[...], preferred_element_type=jnp.float32) + ba_ref[...]
    u = jnp.maximum(u, 0.0)
    u = jnp.dot(u, wb_ref[...], preferred_element_type=jnp.float32) + bb_ref[...]
    o_ref[...] = (
        jnp.dot(u, wfc_ref[...], preferred_element_type=jnp.float32) + bfc_ref[...]
    )


def _row_spec(d):
    return pl.BlockSpec((BLK, d), lambda i: (i, 0))


def _full_spec(r, d):
    return pl.BlockSpec((r, d), lambda i: (0, 0))


_mlp1 = pl.pallas_call(
    _mlp1_body,
    grid=(N_NODES // BLK,),
    in_specs=[
        _row_spec(D_FEAT), _row_spec(D_FEAT), _row_spec(D_FEAT),
        _full_spec(D_FEAT, HIDDEN), _full_spec(1, HIDDEN),
        _full_spec(HIDDEN, HIDDEN), _full_spec(1, HIDDEN),
    ],
    out_specs=_row_spec(HIDDEN),
    out_shape=jax.ShapeDtypeStruct((N_NODES, HIDDEN), jnp.float32),
)

_mlp2 = pl.pallas_call(
    _mlp2_body,
    grid=(N_NODES // BLK,),
    in_specs=[
        _row_spec(HIDDEN), _row_spec(HIDDEN), _row_spec(HIDDEN),
        _full_spec(HIDDEN, HIDDEN), _full_spec(1, HIDDEN),
        _full_spec(HIDDEN, HIDDEN), _full_spec(1, HIDDEN),
        _full_spec(HIDDEN, N_CLASSES), _full_spec(1, N_CLASSES),
    ],
    out_specs=_row_spec(N_CLASSES),
    out_shape=jax.ShapeDtypeStruct((N_NODES, N_CLASSES), jnp.float32),
)


def kernel(x, edge_index, W1a, b1a, W1b, b1b, W2a, b2a, W2b, b2b, Wfc, bfc):
    src = edge_index[0].astype(jnp.int32)
    dst = edge_index[1].astype(jnp.int32)

    a = _sc_aggregate(x, src, dst)
    h1 = _mlp1(x, a[0, :N_NODES], a[1, :N_NODES],
               W1a, b1a.reshape(1, -1), W1b, b1b.reshape(1, -1))
    b = _sc_aggregate(h1, src, dst)
    out = _mlp2(h1, b[0, :N_NODES], b[1, :N_NODES],
                W2a, b2a.reshape(1, -1), W2b, b2b.reshape(1, -1),
                Wfc, bfc.reshape(1, -1))
    return out


# speedup vs baseline: 9.8925x; 2.1599x over previous
import functools

import jax
import jax.numpy as jnp
from jax import lax
from jax.experimental import pallas as pl
from jax.experimental.pallas import tpu as pltpu
from jax.experimental.pallas import tpu_sc as plsc

N_NODES = 10000
N_EDGES = 320000
D_FEAT = 128
HIDDEN = 128
N_CLASSES = 64

NC = 2
NS = 16
NW = NC * NS
EPW = N_EDGES // NW
CE = 80
NCHUNK = EPW // CE
N_PAD = 10240
STRIPE = N_PAD // NS
G = 25
NG = NCHUNK // G

_sc_mesh = plsc.VectorSubcoreMesh(
    core_axis_name="c", subcore_axis_name="s", num_cores=NC, num_subcores=NS
)


@functools.partial(
    pl.kernel,
    out_type=jax.ShapeDtypeStruct((NC, N_PAD, D_FEAT), jnp.float32),
    mesh=_sc_mesh,
    scratch_types=[
        pltpu.VMEM((G, CE), jnp.int32),
        pltpu.VMEM((G, CE), jnp.int32),
        pltpu.VMEM((G, CE), jnp.int32),
        pltpu.VMEM((G, CE), jnp.int32),
        pltpu.VMEM((CE, D_FEAT), jnp.float32),
        pltpu.VMEM((CE, D_FEAT), jnp.float32),
        pltpu.VMEM_SHARED((N_PAD, D_FEAT), jnp.float32),
        pltpu.SemaphoreType.DMA,
        pltpu.SemaphoreType.DMA,
        pltpu.SemaphoreType.DMA,
    ],
)
def _sc_aggregate(x_hbm, src_hbm, dst_hbm, out_hbm, src_v0, src_v1, dst_v0,
                  dst_v1, rows_a, rows_b, aggr_sh, sem_a, sem_b, sem_i):
    c = lax.axis_index("c")
    s = lax.axis_index("s")
    wid = c * NS + s

    idx_waits = [
        pltpu.async_copy(src_hbm.at[wid * NG], src_v0, sem_i),
        pltpu.async_copy(dst_hbm.at[wid * NG], dst_v0, sem_i),
    ]

    zvec = jnp.zeros((16,), jnp.float32)

    def _zb_body(r, carry):
        for j in range(D_FEAT // 16):
            rows_a[r, pl.ds(j * 16, 16)] = zvec
        return carry

    lax.fori_loop(0, CE, _zb_body, 0)
    row0 = s * STRIPE
    for k in range(STRIPE // CE):
        pltpu.sync_copy(rows_a, aggr_sh.at[pl.ds(row0 + k * CE, CE)])
    for w in idx_waits:
        w.wait()
    plsc.subcore_barrier()

    def _wait_gather(buf, sem):
        pltpu.make_async_copy(x_hbm.at[src_v0.at[0]], buf, sem).wait()

    for g in range(NG):
        sv, dv = (src_v0, dst_v0) if g % 2 == 0 else (src_v1, dst_v1)
        if g + 1 < NG:
            nsv, ndv = (src_v0, dst_v0) if (g + 1) % 2 == 0 else (src_v1, dst_v1)
            idx_waits = [
                pltpu.async_copy(src_hbm.at[wid * NG + g + 1], nsv, sem_i),
                pltpu.async_copy(dst_hbm.at[wid * NG + g + 1], ndv, sem_i),
            ]
        pltpu.async_copy(x_hbm.at[sv.at[0]], rows_a, sem_a)

        def _pair_body(i, carry, sv=sv, dv=dv):
            pltpu.async_copy(x_hbm.at[sv.at[2 * i + 1]], rows_b, sem_b)
            _wait_gather(rows_a, sem_a)
            pltpu.sync_copy(rows_a, aggr_sh.at[dv.at[2 * i]], add=True)
            pltpu.async_copy(
                x_hbm.at[sv.at[jnp.minimum(2 * i + 2, G - 1)]], rows_a, sem_a)
            _wait_gather(rows_b, sem_b)
            pltpu.sync_copy(rows_b, aggr_sh.at[dv.at[2 * i + 1]], add=True)
            return carry

        lax.fori_loop(0, G // 2, _pair_body, 0)
        _wait_gather(rows_a, sem_a)
        pltpu.sync_copy(rows_a, aggr_sh.at[dv.at[G - 1]], add=True)
        if g + 1 < NG:
            for w in idx_waits:
                w.wait()
    plsc.subcore_barrier()

    pltpu.sync_copy(
        aggr_sh.at[pl.ds(row0, STRIPE)], out_hbm.at[c, pl.ds(row0, STRIPE)]
    )


BLK = 1000


def _mlp1_body(x_ref, a0_ref, a1_ref, wa_ref, ba_ref, wb_ref, bb_ref, o_ref):
    t = x_ref[...] + a0_ref[...] + a1_ref[...]
    h = jnp.dot(t, wa_ref[...], preferred_element_type=jnp.float32) + ba_ref[...]
    h = jnp.maximum(h, 0.0)
    h = jnp.dot(h, wb_ref[...], preferred_element_type=jnp.float32) + bb_ref[...]
    o_ref[...] = jnp.maximum(h, 0.0)


def _mlp2_body(h_ref, a0_ref, a1_ref, wa_ref, ba_ref, wb_ref, bb_ref,
               wfc_ref, bfc_ref, o_ref):
    t = h_ref[...] + a0_ref[...] + a1_ref[...]
    u = jnp.dot(t, wa_ref[...], preferred_element_type=jnp.float32) + ba_ref[...]
    u = jnp.maximum(u, 0.0)
    u = jnp.dot(u, wb_ref[...], preferred_element_type=jnp.float32) + bb_ref[...]
    o_ref[...] = (
        jnp.dot(u, wfc_ref[...], preferred_element_type=jnp.float32) + bfc_ref[...]
    )


def _row_spec(d):
    return pl.BlockSpec((BLK, d), lambda i: (i, 0))


def _full_spec(r, d):
    return pl.BlockSpec((r, d), lambda i: (0, 0))


_mlp1 = pl.pallas_call(
    _mlp1_body,
    grid=(N_NODES // BLK,),
    in_specs=[
        _row_spec(D_FEAT), _row_spec(D_FEAT), _row_spec(D_FEAT),
        _full_spec(D_FEAT, HIDDEN), _full_spec(1, HIDDEN),
        _full_spec(HIDDEN, HIDDEN), _full_spec(1, HIDDEN),
    ],
    out_specs=_row_spec(HIDDEN),
    out_shape=jax.ShapeDtypeStruct((N_NODES, HIDDEN), jnp.float32),
)

_mlp2 = pl.pallas_call(
    _mlp2_body,
    grid=(N_NODES // BLK,),
    in_specs=[
        _row_spec(HIDDEN), _row_spec(HIDDEN), _row_spec(HIDDEN),
        _full_spec(HIDDEN, HIDDEN), _full_spec(1, HIDDEN),
        _full_spec(HIDDEN, HIDDEN), _full_spec(1, HIDDEN),
        _full_spec(HIDDEN, N_CLASSES), _full_spec(1, N_CLASSES),
    ],
    out_specs=_row_spec(N_CLASSES),
    out_shape=jax.ShapeDtypeStruct((N_NODES, N_CLASSES), jnp.float32),
)


def kernel(x, edge_index, W1a, b1a, W1b, b1b, W2a, b2a, W2b, b2b, Wfc, bfc):
    src = edge_index[0].astype(jnp.int32).reshape(NW * NG, G, CE)
    dst = edge_index[1].astype(jnp.int32).reshape(NW * NG, G, CE)

    a = _sc_aggregate(x, src, dst)
    h1 = _mlp1(x, a[0, :N_NODES], a[1, :N_NODES],
               W1a, b1a.reshape(1, -1), W1b, b1b.reshape(1, -1))
    b = _sc_aggregate(h1, src, dst)
    out = _mlp2(h1, b[0, :N_NODES], b[1, :N_NODES],
                W2a, b2a.reshape(1, -1), W2b, b2b.reshape(1, -1),
                Wfc, bfc.reshape(1, -1))
    return out
